# baseline (speedup 1.0000x reference)
import jax
import jax.numpy as jnp
from jax import lax
from jax.experimental import pallas as pl
from jax.experimental.pallas import tpu as pltpu
from jax.experimental.pallas import tpu_sc as plsc

_NCORES = 2
_NSUB = 16
_NW = _NCORES * _NSUB
_B = 128
_LANES = 16
_DEGW = 16
_TSPREAD = 128


def _mesh():
    return plsc.VectorSubcoreMesh(core_axis_name="core", subcore_axis_name="subcore",
                                  num_cores=_NCORES, num_subcores=_NSUB)


_SC_PARAMS = pltpu.CompilerParams(use_tc_tiling_on_sc=False)


def _build_deg_call(N, D, NP, SHARE, NB):
    TRASH = N

    def body(ep_hbm, z_hbm, ones_hbm, degp_hbm, row_v, col_v, ridx, ones_v,
             deg_acc, sem):
        c = lax.axis_index("core")
        s = lax.axis_index("subcore")
        wid = c * _NSUB + s
        iota = lax.iota(jnp.int32, _LANES)
        pltpu.async_copy(z_hbm, deg_acc.at[pl.ds(s * SHARE, SHARE)], sem).wait()
        pltpu.async_copy(ep_hbm.at[0, wid], row_v, sem).wait()
        pltpu.async_copy(ep_hbm.at[1, wid], col_v, sem).wait()
        pltpu.async_copy(ones_hbm, ones_v, sem).wait()

        @pl.loop(0, NB)
        def _(j):
            @pl.loop(0, _B // _LANES)
            def _(k):
                trash_v = TRASH + ((k * _LANES + iota) & (_TSPREAD - 1))
                r = row_v[j, pl.ds(k * _LANES, _LANES)]
                cc = col_v[j, pl.ds(k * _LANES, _LANES)]
                ridx[j, pl.ds(k * _LANES, _LANES)] = jnp.where(r == cc, trash_v, r)

        plsc.subcore_barrier()

        @pl.loop(0, NB)
        def _(j):
            pltpu.sync_copy(ones_v, deg_acc.at[ridx.at[j]], add=True)

        plsc.subcore_barrier()
        pltpu.async_copy(deg_acc.at[pl.ds(s * SHARE, SHARE)],
                         degp_hbm.at[c, pl.ds(s * SHARE, SHARE)], sem).wait()

    return pl.kernel(
        body,
        out_type=jax.ShapeDtypeStruct((_NCORES, NP, _DEGW), jnp.float32),
        mesh=_mesh(),
        compiler_params=_SC_PARAMS,
        scratch_types=[
            pltpu.VMEM((NB, _B), jnp.int32),
            pltpu.VMEM((NB, _B), jnp.int32),
            pltpu.VMEM((NB, _B), jnp.int32),
            pltpu.VMEM((_B, _DEGW), jnp.float32),
            pltpu.VMEM_SHARED((NP, _DEGW), jnp.float32),
            pltpu.SemaphoreType.DMA,
        ],
    )


def _build_agg_call(N, D, NP, SHARE, NB):
    TRASH = N
    DH = D // _NCORES
    NB2 = _NCORES * NB

    R = 4

    def body(xsp_hbm, ep_hbm, z_hbm, agg_hbm, row_v, col_v, bufs,
             out_acc, sem, *ring_sems):
        gsem = ring_sems[:R]
        ssem = ring_sems[R:]
        c = lax.axis_index("core")
        s = lax.axis_index("subcore")
        pltpu.async_copy(z_hbm, out_acc.at[pl.ds(s * SHARE, SHARE)], sem).wait()
        pltpu.async_copy(ep_hbm.at[0, 2 * s], row_v.at[pl.ds(0, NB)], sem).wait()
        pltpu.async_copy(ep_hbm.at[0, 2 * s + 1], row_v.at[pl.ds(NB, NB)], sem).wait()
        pltpu.async_copy(ep_hbm.at[1, 2 * s], col_v.at[pl.ds(0, NB)], sem).wait()
        pltpu.async_copy(ep_hbm.at[1, 2 * s + 1], col_v.at[pl.ds(NB, NB)], sem).wait()

        iota = lax.iota(jnp.int32, _LANES)

        @pl.loop(0, NB2)
        def _(j):
            @pl.loop(0, _B // _LANES)
            def _(k):
                trash_v = TRASH + ((k * _LANES + iota) & (_TSPREAD - 1))
                r = row_v[j, pl.ds(k * _LANES, _LANES)]
                cc = col_v[j, pl.ds(k * _LANES, _LANES)]
                col_v[j, pl.ds(k * _LANES, _LANES)] = jnp.where(r == cc, trash_v, cc)

        plsc.subcore_barrier()

        xs_half = xsp_hbm.at[c]

        def issue_gather(j, b):
            pltpu.async_copy(xs_half.at[row_v.at[j]], bufs.at[b], gsem[b])

        def issue_scatter(j, b):
            pltpu.async_copy(bufs.at[b], out_acc.at[col_v.at[j]],
                             ssem[b], add=True)

        def wait_gather(b):
            pltpu.make_async_copy(xs_half.at[pl.ds(0, _B)], bufs.at[b],
                                  gsem[b]).wait()

        def wait_scatter(b):
            pltpu.make_async_copy(xs_half.at[pl.ds(0, _B)], bufs.at[b],
                                  ssem[b]).wait()

        for b in range(R):
            issue_gather(b, b)

        @pl.loop(0, NB2, step=R)
        def _(j):
            for b in range(R):
                jc = j + b
                wait_gather(b)
                issue_scatter(jc, b)

                @pl.when(jc + R < NB2)
                def _():
                    wait_scatter(b)
                    issue_gather(jc + R, b)

        for b in range(R):
            wait_scatter(b)
        plsc.subcore_barrier()
        pltpu.async_copy(out_acc.at[pl.ds(s * SHARE, SHARE)],
                         agg_hbm.at[c, pl.ds(s * SHARE, SHARE)], sem).wait()

    return pl.kernel(
        body,
        out_type=jax.ShapeDtypeStruct((_NCORES, NP, DH), jnp.float32),
        mesh=_mesh(),
        compiler_params=_SC_PARAMS,
        scratch_types=[
            pltpu.VMEM((NB2, _B), jnp.int32),
            pltpu.VMEM((NB2, _B), jnp.int32),
            pltpu.VMEM((R, _B, DH), jnp.float32),
            pltpu.VMEM_SHARED((NP, DH), jnp.float32),
            pltpu.SemaphoreType.DMA,
            *([pltpu.SemaphoreType.DMA] * (2 * R)),
        ],
    )


def _prep_call(N, D):
    DH = D // _NCORES

    def body(x_ref, degp_ref, xsp_ref, dinv_ref):
        deg = degp_ref[0, :N, 0:1] + degp_ref[1, :N, 0:1] + 1.0
        dinv = lax.rsqrt(deg)
        dinv_ref[...] = dinv
        xsp_ref[0] = x_ref[:, :DH] * dinv
        xsp_ref[1] = x_ref[:, DH:] * dinv

    return pl.pallas_call(
        body,
        out_shape=(
            jax.ShapeDtypeStruct((_NCORES, N, DH), jnp.float32),
            jax.ShapeDtypeStruct((N, 1), jnp.float32),
        ),
    )


def _final_call(N, D):
    DH = D // _NCORES

    def body(agg_ref, xsp_ref, dinv_ref, out_ref):
        a0 = agg_ref[0, :N, :] + xsp_ref[0]
        a1 = agg_ref[1, :N, :] + xsp_ref[1]
        dinv = dinv_ref[...]
        out_ref[...] = jnp.concatenate([dinv * a0, dinv * a1], axis=1)

    return pl.pallas_call(
        body,
        out_shape=jax.ShapeDtypeStruct((N, D), jnp.float32),
    )


def kernel(x, edge_index):
    N, D = x.shape
    E = edge_index.shape[1]
    EPT = pl.cdiv(E, _NW)
    NB = pl.cdiv(EPT, _B)
    if NB % 2:
        NB += 1
    NP = N + max(_B, _TSPREAD)
    NP += (-NP) % (_NSUB * 8)
    SHARE = NP // _NSUB

    EPAD = _NW * NB * _B
    pad_vals = (jnp.arange(EPAD - E, dtype=jnp.int32) % N)[None, :]
    ep = jnp.concatenate(
        [edge_index, jnp.broadcast_to(pad_vals, (2, EPAD - E))], axis=1
    ).reshape(2, _NW, NB, _B)

    z16 = jnp.zeros((SHARE, _DEGW), jnp.float32)
    ones16 = jnp.ones((_B, _DEGW), jnp.float32)
    zh = jnp.zeros((SHARE, D // _NCORES), jnp.float32)

    degp = _build_deg_call(N, D, NP, SHARE, NB)(ep, z16, ones16)
    xsp, dinv = _prep_call(N, D)(x, degp)
    agg = _build_agg_call(N, D, NP, SHARE, NB)(xsp, ep, zh)
    return _final_call(N, D)(agg, xsp, dinv)


# speedup vs baseline: 1.0221x; 1.0221x over previous
import jax
import jax.numpy as jnp
from jax import lax
from jax.experimental import pallas as pl
from jax.experimental.pallas import tpu as pltpu
from jax.experimental.pallas import tpu_sc as plsc

_NCORES = 2
_NSUB = 16
_NW = _NCORES * _NSUB
_B = 128
_LANES = 16
_DEGW = 16
_TSPREAD = 128


def _mesh():
    return plsc.VectorSubcoreMesh(core_axis_name="core", subcore_axis_name="subcore",
                                  num_cores=_NCORES, num_subcores=_NSUB)


_SC_PARAMS = pltpu.CompilerParams(use_tc_tiling_on_sc=False)


def _build_deg_call(N, D, NP, SHARE, NB):
    TRASH = N

    def body(ep_hbm, z_hbm, ones_hbm, degp_hbm, row_v, col_v, ridx, ones_v,
             deg_acc, sem, ssem):
        c = lax.axis_index("core")
        s = lax.axis_index("subcore")
        wid = c * _NSUB + s
        iota = lax.iota(jnp.int32, _LANES)
        pltpu.async_copy(z_hbm, deg_acc.at[pl.ds(s * SHARE, SHARE)], sem).wait()
        pltpu.async_copy(ep_hbm.at[0, wid], row_v, sem).wait()
        pltpu.async_copy(ep_hbm.at[1, wid], col_v, sem).wait()
        pltpu.async_copy(ones_hbm, ones_v, sem).wait()

        plsc.subcore_barrier()

        @pl.loop(0, NB)
        def _(j):
            @pl.loop(0, _B // _LANES)
            def _(k):
                trash_v = TRASH + ((k * _LANES + iota) & (_TSPREAD - 1))
                r = row_v[j, pl.ds(k * _LANES, _LANES)]
                cc = col_v[j, pl.ds(k * _LANES, _LANES)]
                ridx[j, pl.ds(k * _LANES, _LANES)] = jnp.where(r == cc, trash_v, r)

            pltpu.async_copy(ones_v, deg_acc.at[ridx.at[j]], ssem, add=True)

        @pl.loop(0, NB)
        def _(j):
            pltpu.make_async_copy(ones_hbm, ones_v, ssem).wait()

        plsc.subcore_barrier()
        pltpu.async_copy(deg_acc.at[pl.ds(s * SHARE, SHARE)],
                         degp_hbm.at[c, pl.ds(s * SHARE, SHARE)], sem).wait()

    return pl.kernel(
        body,
        out_type=jax.ShapeDtypeStruct((_NCORES, NP, _DEGW), jnp.float32),
        mesh=_mesh(),
        compiler_params=_SC_PARAMS,
        scratch_types=[
            pltpu.VMEM((NB, _B), jnp.int32),
            pltpu.VMEM((NB, _B), jnp.int32),
            pltpu.VMEM((NB, _B), jnp.int32),
            pltpu.VMEM((_B, _DEGW), jnp.float32),
            pltpu.VMEM_SHARED((NP, _DEGW), jnp.float32),
            pltpu.SemaphoreType.DMA,
            pltpu.SemaphoreType.DMA,
        ],
    )


def _build_agg_call(N, D, NP, SHARE, NB):
    TRASH = N
    DH = D // _NCORES
    NB2 = _NCORES * NB

    R = 4

    LAST = N - (_NSUB - 1) * SHARE

    def body(xsp_hbm, ep_hbm, z_hbm, dinv_hbm, agg_hbm, row_v, col_v, bufs,
             dvv, out_acc, sem, *ring_sems):
        gsem = ring_sems[:R]
        ssem = ring_sems[R:]
        c = lax.axis_index("core")
        s = lax.axis_index("subcore")
        base = s * SHARE
        @pl.when(s == _NSUB - 1)
        def _():
            pltpu.async_copy(z_hbm, out_acc.at[pl.ds(base, SHARE)], sem).wait()
            pltpu.async_copy(xsp_hbm.at[c, pl.ds(base, LAST)],
                             out_acc.at[pl.ds(base, LAST)], sem).wait()

        @pl.when(s < _NSUB - 1)
        def _():
            pltpu.async_copy(xsp_hbm.at[c, pl.ds(base, SHARE)],
                             out_acc.at[pl.ds(base, SHARE)], sem).wait()
        pltpu.async_copy(ep_hbm.at[0, 2 * s], row_v.at[pl.ds(0, NB)], sem).wait()
        pltpu.async_copy(ep_hbm.at[0, 2 * s + 1], row_v.at[pl.ds(NB, NB)], sem).wait()
        pltpu.async_copy(ep_hbm.at[1, 2 * s], col_v.at[pl.ds(0, NB)], sem).wait()
        pltpu.async_copy(ep_hbm.at[1, 2 * s + 1], col_v.at[pl.ds(NB, NB)], sem).wait()

        iota = lax.iota(jnp.int32, _LANES)

        @pl.loop(0, NB2)
        def _(j):
            @pl.loop(0, _B // _LANES)
            def _(k):
                trash_v = TRASH + ((k * _LANES + iota) & (_TSPREAD - 1))
                r = row_v[j, pl.ds(k * _LANES, _LANES)]
                cc = col_v[j, pl.ds(k * _LANES, _LANES)]
                col_v[j, pl.ds(k * _LANES, _LANES)] = jnp.where(r == cc, trash_v, cc)

        plsc.subcore_barrier()

        xs_half = xsp_hbm.at[c]

        def issue_gather(j, b):
            pltpu.async_copy(xs_half.at[row_v.at[j]], bufs.at[b], gsem[b])

        def issue_scatter(j, b):
            pltpu.async_copy(bufs.at[b], out_acc.at[col_v.at[j]],
                             ssem[b], add=True)

        def wait_gather(b):
            pltpu.make_async_copy(xs_half.at[pl.ds(0, _B)], bufs.at[b],
                                  gsem[b]).wait()

        def wait_scatter(b):
            pltpu.make_async_copy(xs_half.at[pl.ds(0, _B)], bufs.at[b],
                                  ssem[b]).wait()

        for b in range(R):
            issue_gather(b, b)

        @pl.loop(0, NB2, step=R)
        def _(j):
            for b in range(R):
                jc = j + b
                wait_gather(b)
                issue_scatter(jc, b)

                @pl.when(jc + R < NB2)
                def _():
                    wait_scatter(b)
                    issue_gather(jc + R, b)

        for b in range(R):
            wait_scatter(b)
        plsc.subcore_barrier()
        @pl.when(s == _NSUB - 1)
        def _():
            pltpu.async_copy(dinv_hbm.at[pl.ds(base, LAST)],
                             dvv.at[pl.ds(0, LAST)], sem).wait()

        @pl.when(s < _NSUB - 1)
        def _():
            pltpu.async_copy(dinv_hbm.at[pl.ds(base, SHARE)], dvv, sem).wait()

        @pl.loop(0, SHARE, step=_B)
        def _(r0):
            pltpu.async_copy(out_acc.at[pl.ds(base + r0, _B)], bufs.at[0],
                             sem).wait()

            @pl.loop(0, _B)
            def _(rr):
                d = dvv[r0 + rr, :]
                for k in range(DH // _LANES):
                    sl = pl.ds(k * _LANES, _LANES)
                    bufs[0, rr, sl] = bufs[0, rr, sl] * d

            pltpu.async_copy(bufs.at[0], agg_hbm.at[c, pl.ds(base + r0, _B)],
                             sem).wait()

    return pl.kernel(
        body,
        out_type=jax.ShapeDtypeStruct((_NCORES, NP, DH), jnp.float32),
        mesh=_mesh(),
        compiler_params=_SC_PARAMS,
        scratch_types=[
            pltpu.VMEM((NB2, _B), jnp.int32),
            pltpu.VMEM((NB2, _B), jnp.int32),
            pltpu.VMEM((R, _B, DH), jnp.float32),
            pltpu.VMEM((SHARE, _LANES), jnp.float32),
            pltpu.VMEM_SHARED((NP, DH), jnp.float32),
            pltpu.SemaphoreType.DMA,
            *([pltpu.SemaphoreType.DMA] * (2 * R)),
        ],
    )


def _prep_call(N, D):
    DH = D // _NCORES

    def body(x_ref, degp_ref, xsp_ref, dinv_ref):
        deg = degp_ref[0, :N, 0:1] + degp_ref[1, :N, 0:1] + 1.0
        dinv = lax.rsqrt(deg)
        dinv_ref[...] = jnp.broadcast_to(dinv, (N, _LANES))
        xsp_ref[0] = x_ref[:, :DH] * dinv
        xsp_ref[1] = x_ref[:, DH:] * dinv

    return pl.pallas_call(
        body,
        out_shape=(
            jax.ShapeDtypeStruct((_NCORES, N, DH), jnp.float32),
            jax.ShapeDtypeStruct((N, _LANES), jnp.float32),
        ),
    )


def _final_call(N, D):
    DH = D // _NCORES

    def body(agg_ref, xsp_ref, dinv_ref, out_ref):
        a0 = agg_ref[0, :N, :] + xsp_ref[0]
        a1 = agg_ref[1, :N, :] + xsp_ref[1]
        dinv = dinv_ref[...]
        out_ref[...] = jnp.concatenate([dinv * a0, dinv * a1], axis=1)

    return pl.pallas_call(
        body,
        out_shape=jax.ShapeDtypeStruct((N, D), jnp.float32),
    )


def kernel(x, edge_index):
    N, D = x.shape
    E = edge_index.shape[1]
    EPT = pl.cdiv(E, _NW)
    NB = pl.cdiv(EPT, _B)
    if NB % 2:
        NB += 1
    NP = N + max(_B, _TSPREAD)
    NP += (-NP) % (_NSUB * 8)
    SHARE = NP // _NSUB

    EPAD = _NW * NB * _B
    pad_vals = (jnp.arange(EPAD - E, dtype=jnp.int32) % N)[None, :]
    ep = jnp.concatenate(
        [edge_index, jnp.broadcast_to(pad_vals, (2, EPAD - E))], axis=1
    ).reshape(2, _NW, NB, _B)

    z16 = jnp.zeros((SHARE, _DEGW), jnp.float32)
    ones16 = jnp.ones((_B, _DEGW), jnp.float32)
    zh = jnp.zeros((SHARE, D // _NCORES), jnp.float32)

    degp = _build_deg_call(N, D, NP, SHARE, NB)(ep, z16, ones16)
    xsp, dinv = _prep_call(N, D)(x, degp)
    agg = _build_agg_call(N, D, NP, SHARE, NB)(xsp, ep, zh, dinv)
    return jnp.concatenate([agg[0, :N], agg[1, :N]], axis=1)
